# baseline (speedup 1.0000x reference)
import functools

import jax
import jax.numpy as jnp
from jax import lax
from jax.experimental import pallas as pl
from jax.experimental.pallas import tpu as pltpu
from jax.experimental.pallas import tpu_sc as plsc

N = 10000
E = 320000
F = 128
O = 64
NC = 2
NS = 16
NW = NC * NS
L = 16

BB = 128
NB = 79
EPT = NB * BB
E_PAD = NW * EPT
N_TAB = 10240
RPS = N_TAB // NS

_mesh = plsc.VectorSubcoreMesh(
    core_axis_name="c", subcore_axis_name="s", num_cores=NC, num_subcores=NS)


def _bcast(v, j):
    return lax.gather(
        v, jnp.full((L, 1), j, jnp.int32),
        lax.GatherDimensionNumbers(
            offset_dims=(), collapsed_slice_dims=(0,), start_index_map=(0,)),
        (1,), mode=lax.GatherScatterMode.PROMISE_IN_BOUNDS)


def _deg_body(dstf_hbm, zeros_hbm, out_hbm, dstf, tab, red, stage):
    c = lax.axis_index("c")
    s = lax.axis_index("s")
    w = s * NC + c
    pltpu.sync_copy(dstf_hbm.at[w], dstf)
    pltpu.sync_copy(zeros_hbm, tab)
    ones = jnp.full((L,), 1.0, jnp.float32)

    def body(i, carry):
        idx = dstf[pl.ds(i * L, L)]
        plsc.addupdate_scatter(tab, [idx], ones)
        return carry

    lax.fori_loop(0, EPT // L, body, 0)
    pltpu.sync_copy(tab, stage.at[s])
    plsc.subcore_barrier()
    pltpu.sync_copy(stage.at[:, pl.ds(s * RPS, RPS)], red)

    def rbody(k, carry):
        acc = jnp.zeros((L,), jnp.float32)
        for t in range(NS):
            acc = acc + red[t, pl.ds(k * L, L)]
        tab[pl.ds(k * L, L)] = acc
        return carry

    lax.fori_loop(0, RPS // L, rbody, 0)
    pltpu.sync_copy(tab.at[pl.ds(0, RPS)],
                    out_hbm.at[c].at[pl.ds(s * RPS, RPS)])


_deg_call = functools.partial(
    pl.kernel,
    out_type=jax.ShapeDtypeStruct((NC, N_TAB), jnp.float32),
    mesh=_mesh,
    compiler_params=pltpu.CompilerParams(needs_layout_passes=False),
    scratch_types=[
        pltpu.VMEM((EPT,), jnp.int32),
        pltpu.VMEM((N_TAB,), jnp.float32),
        pltpu.VMEM((NS, RPS), jnp.float32),
        pltpu.VMEM_SHARED((NS, N_TAB), jnp.float32),
    ],
)(_deg_body)


FH = F // 2


def _gcn_body(tab_hbm, src2_hbm, dst2_hbm, zrow_hbm, out_hbm,
              srcv, dstv, buf0, buf1, g0, g1, s0, s1, tabsh, acc):
    c = lax.axis_index("c")
    s = lax.axis_index("s")
    w = s * NC + c
    pltpu.sync_copy(src2_hbm.at[w], srcv)
    pltpu.sync_copy(dst2_hbm.at[w], dstv)
    base = s * RPS

    for half in range(2):
        for t in range(5):
            rows = pl.ds(base + t * BB, BB)
            pltpu.sync_copy(tab_hbm.at[rows, pl.ds(half * FH, FH)], buf0)
            pltpu.sync_copy(buf0, tabsh.at[rows])
            pltpu.sync_copy(zrow_hbm, buf1)
            pltpu.sync_copy(buf1, acc.at[rows])
        plsc.subcore_barrier()

        pltpu.async_copy(tabsh.at[srcv.at[0]], buf0, g0)

        def body(b, carry):
            even = lax.rem(b, 2) == 0

            @pl.when(jnp.logical_and(even, b > 0))
            def _():
                pltpu.make_async_copy(buf1, acc.at[dstv.at[b]], s1).wait()

            @pl.when(jnp.logical_and(even, b + 1 < NB))
            def _():
                pltpu.async_copy(tabsh.at[srcv.at[b + 1]], buf1, g1)

            @pl.when(jnp.logical_not(even))
            def _():
                pltpu.make_async_copy(buf0, acc.at[dstv.at[b]], s0).wait()
                pltpu.async_copy(tabsh.at[srcv.at[b + 1]], buf0, g0)

            @pl.when(even)
            def _():
                pltpu.make_async_copy(tabsh.at[srcv.at[b]], buf0, g0).wait()
                pltpu.async_copy(buf0, acc.at[dstv.at[b]], s0, add=True)

            @pl.when(jnp.logical_not(even))
            def _():
                pltpu.make_async_copy(tabsh.at[srcv.at[b]], buf1, g1).wait()
                pltpu.async_copy(buf1, acc.at[dstv.at[b]], s1, add=True)

            return carry

        lax.fori_loop(0, NB, body, 0)
        pltpu.make_async_copy(buf0, acc.at[dstv.at[NB - 1]], s0).wait()
        plsc.subcore_barrier()
        for t in range(5):
            rows = pl.ds(base + t * BB, BB)
            pltpu.sync_copy(acc.at[rows], buf0)
            pltpu.sync_copy(buf0,
                            out_hbm.at[c].at[rows, pl.ds(half * FH, FH)])
        if half == 0:
            plsc.subcore_barrier()


_gcn_call = functools.partial(
    pl.kernel,
    out_type=jax.ShapeDtypeStruct((NC, N_TAB, F), jnp.float32),
    mesh=_mesh,
    compiler_params=pltpu.CompilerParams(
        needs_layout_passes=False, use_tc_tiling_on_sc=False),
    scratch_types=[
        pltpu.VMEM((NB, BB), jnp.int32),
        pltpu.VMEM((NB, BB), jnp.int32),
        pltpu.VMEM((BB, FH), jnp.float32),
        pltpu.VMEM((BB, FH), jnp.float32),
        pltpu.SemaphoreType.DMA,
        pltpu.SemaphoreType.DMA,
        pltpu.SemaphoreType.DMA,
        pltpu.SemaphoreType.DMA,
        pltpu.VMEM_SHARED((N_TAB, FH), jnp.float32),
        pltpu.VMEM_SHARED((N_TAB, FH), jnp.float32),
    ],
)(_gcn_body)


def _gat_body(gtab_hbm, asrc_hbm, adst_hbm, maxs_hbm,
              src2_hbm, dst2_hbm, zeros_hbm, zrow_hbm,
              out_hbm, esum_hbm, ex_hbm,
              srcv, dstv, asv, adv, esv, exv,
              bufa, bufb, bufc, ga, gb, gc, sa, sb, sc_, maxv, acc):
    bufs = [bufa, bufb, bufc]
    gsems = [ga, gb, gc]
    ssems = [sa, sb, sc_]
    c = lax.axis_index("c")
    s = lax.axis_index("s")
    w = s * NC + c
    pltpu.sync_copy(src2_hbm.at[w], srcv)
    pltpu.sync_copy(dst2_hbm.at[w], dstv)
    pltpu.sync_copy(asrc_hbm, asv)
    pltpu.sync_copy(adst_hbm, adv)
    pltpu.sync_copy(zeros_hbm, esv)
    pltpu.sync_copy(maxs_hbm, maxv)
    pltpu.sync_copy(zrow_hbm, bufs[0])
    base = s * RPS
    for t in range(5):
        pltpu.sync_copy(bufs[0], acc.at[pl.ds(base + t * BB, BB)])
    plsc.subcore_barrier()

    mx = maxv[...]

    def compute_ex(b):
        boff = b * BB
        for k in range(BB // L):
            sidx = srcv[b, pl.ds(k * L, L)]
            didx = dstv[b, pl.ds(k * L, L)]
            a_s = plsc.load_gather(asv, [sidx])
            a_d = plsc.load_gather(adv, [didx])
            e = a_s + a_d
            e = jnp.where(e >= 0.0, e, e * 0.2)
            m = mx + a_d
            m = jnp.where(m >= 0.0, m, m * 0.2)
            ex = jnp.exp(e - m)
            exv[pl.ds(boff + k * L, L)] = ex
            plsc.addupdate_scatter(esv, [didx], ex)

    def scale_rows(bf, b):
        boff = b * BB

        def grp(gi, carry2):
            exk = exv[pl.ds(boff + gi * L, L)]
            for j in range(L):
                exb = _bcast(exk, j)
                r = gi * L + j
                for k in range(O // L):
                    bf[r, pl.ds(k * L, L)] = bf[r, pl.ds(k * L, L)] * exb
            return carry2

        lax.fori_loop(0, BB // L, grp, 0)

    NBUF = 3
    QN = NB // NBUF
    TAIL = NB - QN * NBUF

    for u in range(NBUF):
        pltpu.async_copy(gtab_hbm.at[srcv.at[u]], bufs[u], gsems[u])

    def process(u, b):
        pltpu.make_async_copy(gtab_hbm.at[srcv.at[b]], bufs[u],
                              gsems[u]).wait()
        compute_ex(b)
        scale_rows(bufs[u], b)
        pltpu.async_copy(bufs[u], acc.at[dstv.at[b]], ssems[u], add=True)

    def round_body(q, carry):
        b0 = q * NBUF
        for u in range(NBUF):
            process(u, b0 + u)
        for u in range(NBUF):
            nxt = b0 + NBUF + u

            @pl.when(nxt < NB)
            def _():
                pltpu.make_async_copy(bufs[u], acc.at[dstv.at[b0 + u]],
                                      ssems[u]).wait()
                pltpu.async_copy(gtab_hbm.at[srcv.at[nxt]], bufs[u],
                                 gsems[u])
        return carry

    lax.fori_loop(0, QN, round_body, 0)
    for u in range(TAIL):
        process(u, QN * NBUF + u)
    for u in range(TAIL):
        pltpu.make_async_copy(bufs[u], acc.at[dstv.at[0]], ssems[u]).wait()
    for u in range(TAIL, NBUF):
        pltpu.make_async_copy(bufs[u], acc.at[dstv.at[0]], ssems[u]).wait()
    plsc.subcore_barrier()
    pltpu.sync_copy(esv, esum_hbm.at[w])
    pltpu.sync_copy(exv, ex_hbm.at[w])
    for t in range(5):
        pltpu.sync_copy(acc.at[pl.ds(base + t * BB, BB)], bufs[0])
        pltpu.sync_copy(bufs[0], out_hbm.at[c].at[pl.ds(base + t * BB, BB)])


_gat_call = functools.partial(
    pl.kernel,
    out_type=(
        jax.ShapeDtypeStruct((NC, N_TAB, O), jnp.float32),
        jax.ShapeDtypeStruct((NW, N_TAB), jnp.float32),
        jax.ShapeDtypeStruct((NW, EPT), jnp.float32),
    ),
    mesh=_mesh,
    compiler_params=pltpu.CompilerParams(
        needs_layout_passes=False, use_tc_tiling_on_sc=False),
    scratch_types=[
        pltpu.VMEM((NB, BB), jnp.int32),
        pltpu.VMEM((NB, BB), jnp.int32),
        pltpu.VMEM((N_TAB,), jnp.float32),
        pltpu.VMEM((N_TAB,), jnp.float32),
        pltpu.VMEM((N_TAB,), jnp.float32),
        pltpu.VMEM((EPT,), jnp.float32),
        pltpu.VMEM((BB, O), jnp.float32),
        pltpu.VMEM((BB, O), jnp.float32),
        pltpu.VMEM((BB, O), jnp.float32),
        pltpu.SemaphoreType.DMA,
        pltpu.SemaphoreType.DMA,
        pltpu.SemaphoreType.DMA,
        pltpu.SemaphoreType.DMA,
        pltpu.SemaphoreType.DMA,
        pltpu.SemaphoreType.DMA,
        pltpu.VMEM((L,), jnp.float32),
        pltpu.VMEM_SHARED((N_TAB, O), jnp.float32),
    ],
)(_gat_body)


def _alpha_body(ex_hbm, dstf_hbm, rec_hbm, out_hbm, exv, dstf, recv, av):
    w = lax.axis_index("s") * NC + lax.axis_index("c")
    pltpu.sync_copy(ex_hbm.at[w], exv)
    pltpu.sync_copy(dstf_hbm.at[w], dstf)
    pltpu.sync_copy(rec_hbm, recv)

    def body(i, carry):
        off = i * L
        didx = dstf[pl.ds(off, L)]
        r = plsc.load_gather(recv, [didx])
        av[pl.ds(off, L)] = exv[pl.ds(off, L)] * r
        return carry

    lax.fori_loop(0, EPT // L, body, 0)
    pltpu.sync_copy(av, out_hbm.at[w])


_alpha_call = functools.partial(
    pl.kernel,
    out_type=jax.ShapeDtypeStruct((NW, EPT), jnp.float32),
    mesh=_mesh,
    compiler_params=pltpu.CompilerParams(needs_layout_passes=False),
    scratch_types=[
        pltpu.VMEM((EPT,), jnp.float32),
        pltpu.VMEM((EPT,), jnp.int32),
        pltpu.VMEM((N_TAB,), jnp.float32),
        pltpu.VMEM((EPT,), jnp.float32),
    ],
)(_alpha_body)


def _t1_body(x_ref, w1_ref, degp_ref, h0p_ref, dinv_ref):
    deg = degp_ref[0, :] + degp_ref[1, :] + 1.0
    dinv = lax.rsqrt(deg)[:, None]
    h0 = jnp.dot(x_ref[...], w1_ref[...], preferred_element_type=jnp.float32)
    h0p_ref[pl.ds(0, N), :] = h0 * dinv[:N]
    h0p_ref[pl.ds(N, N_TAB - N), :] = jnp.zeros((N_TAB - N, F), jnp.float32)
    dinv_ref[...] = dinv


def _t1(x, W1, degp):
    return pl.pallas_call(
        _t1_body,
        out_shape=[
            jax.ShapeDtypeStruct((N_TAB, F), jnp.float32),
            jax.ShapeDtypeStruct((N_TAB, 1), jnp.float32),
        ],
    )(x, W1, degp)


def _t2_body(p_ref, h0p_ref, dinv_ref, b1_ref, w2_ref, as_ref,
             ad_ref, g_ref, asrc_ref, adst_ref, exs_ref, maxs_ref):
    S = p_ref[0] + p_ref[1] + h0p_ref[...]
    h = jnp.maximum(dinv_ref[...] * S + b1_ref[...], 0.0)
    g = jnp.dot(h, w2_ref[...], preferred_element_type=jnp.float32)
    asrc = jnp.dot(g, as_ref[...], preferred_element_type=jnp.float32)
    adst = jnp.dot(g, ad_ref[...], preferred_element_type=jnp.float32)
    maxs = jnp.max(asrc)
    m = maxs + adst
    m = jnp.where(m >= 0.0, m, m * 0.2)
    e = asrc + adst
    e = jnp.where(e >= 0.0, e, e * 0.2)
    g_ref[...] = g
    asrc_ref[...] = asrc[:, 0]
    adst_ref[...] = adst[:, 0]
    exs_ref[...] = jnp.exp(e - m)
    maxs_ref[...] = jnp.full((1, L), maxs, jnp.float32)


def _t2(p, h0p, dinv, b1, W2, att_src, att_dst):
    return pl.pallas_call(
        _t2_body,
        out_shape=[
            jax.ShapeDtypeStruct((N_TAB, O), jnp.float32),
            jax.ShapeDtypeStruct((N_TAB,), jnp.float32),
            jax.ShapeDtypeStruct((N_TAB,), jnp.float32),
            jax.ShapeDtypeStruct((N_TAB, 1), jnp.float32),
            jax.ShapeDtypeStruct((1, L), jnp.float32),
        ],
    )(p, h0p, dinv, b1, W2, att_src, att_dst)


def _t3_body(pg_ref, esump_ref, exs_ref, g_ref, b2_ref,
             out_ref, aself_ref, rec_ref):
    exs = exs_ref[...]
    esum = jnp.sum(esump_ref[...], axis=0)[:, None] + exs
    rec = 1.0 / (esum + 1e-16)
    g = g_ref[...]
    full = (pg_ref[0] + pg_ref[1] + exs * g) * rec + b2_ref[...]
    out_ref[...] = full[:N]
    aself_ref[...] = (exs * rec)[:N, 0]
    rec_ref[...] = rec[:, 0]


def _t3(pg, esump, exs, g, b2):
    return pl.pallas_call(
        _t3_body,
        out_shape=[
            jax.ShapeDtypeStruct((N, O), jnp.float32),
            jax.ShapeDtypeStruct((N,), jnp.float32),
            jax.ShapeDtypeStruct((N_TAB,), jnp.float32),
        ],
    )(pg, esump, exs, g, b2)


def kernel(x, edge_index, W1, b1, W2, att_src, att_dst, b2):
    src = edge_index[0]
    dst = edge_index[1]
    pad = jnp.full((E_PAD - E,), N, jnp.int32)
    pad_f = jnp.full((E_PAD - E,), N + 1, jnp.int32)
    srcp = jnp.concatenate([src, pad])
    dstp = jnp.concatenate([dst, pad])
    srcf = jnp.concatenate([src, pad_f]).reshape(NW, EPT)
    dstf = jnp.concatenate([dst, pad_f]).reshape(NW, EPT)
    src2 = srcp.reshape(NW, NB, BB)
    dst2 = dstp.reshape(NW, NB, BB)
    zeros_nt = jnp.zeros((N_TAB,), jnp.float32)
    zrow_f = jnp.zeros((BB, FH), jnp.float32)
    zrow_o = jnp.zeros((BB, O), jnp.float32)

    degp = _deg_call(dstf, zeros_nt)
    h0p, dinv = _t1(x, W1, degp)
    P = _gcn_call(h0p, src2, dst2, zrow_f)
    g, asrc, adst, exs, maxs = _t2(P, h0p, dinv, b1.reshape(1, F), W2,
                                   att_src.reshape(O, 1), att_dst.reshape(O, 1))
    Pg, esump, exe = _gat_call(
        g, asrc, adst, maxs.reshape(L),
        src2, dst2, zeros_nt, zrow_o)
    out, aself, rec = _t3(Pg, esump, exs, g, b2.reshape(1, O))
    alpha_e = _alpha_call(exe, dstf, rec).reshape(E_PAD)[:E]
    alpha = jnp.concatenate([alpha_e, aself])
    ar = jnp.arange(N, dtype=edge_index.dtype)
    ei_full = jnp.stack([jnp.concatenate([src, ar]), jnp.concatenate([dst, ar])])
    return (out, (ei_full, alpha))


# speedup vs baseline: 1.0784x; 1.0784x over previous
import functools

import jax
import jax.numpy as jnp
from jax import lax
from jax.experimental import pallas as pl
from jax.experimental.pallas import tpu as pltpu
from jax.experimental.pallas import tpu_sc as plsc

N = 10000
E = 320000
F = 128
O = 64
NC = 2
NS = 16
NW = NC * NS
L = 16

BB = 128
NB = 79
EPT = NB * BB
E_PAD = NW * EPT
N_TAB = 10240
RPS = N_TAB // NS

_mesh = plsc.VectorSubcoreMesh(
    core_axis_name="c", subcore_axis_name="s", num_cores=NC, num_subcores=NS)


def _bcast(v, j):
    return lax.gather(
        v, jnp.full((L, 1), j, jnp.int32),
        lax.GatherDimensionNumbers(
            offset_dims=(), collapsed_slice_dims=(0,), start_index_map=(0,)),
        (1,), mode=lax.GatherScatterMode.PROMISE_IN_BOUNDS)


def _deg_body(dstf_hbm, zeros_hbm, out_hbm, dstf, tab, red, stage):
    c = lax.axis_index("c")
    s = lax.axis_index("s")
    w = s * NC + c
    pltpu.sync_copy(dstf_hbm.at[w], dstf)
    pltpu.sync_copy(zeros_hbm, tab)
    ones = jnp.full((L,), 1.0, jnp.float32)

    def body(i, carry):
        idx = dstf[pl.ds(i * L, L)]
        plsc.addupdate_scatter(tab, [idx], ones)
        return carry

    lax.fori_loop(0, EPT // L, body, 0)
    pltpu.sync_copy(tab, stage.at[s])
    plsc.subcore_barrier()
    pltpu.sync_copy(stage.at[:, pl.ds(s * RPS, RPS)], red)

    def rbody(k, carry):
        acc = jnp.zeros((L,), jnp.float32)
        for t in range(NS):
            acc = acc + red[t, pl.ds(k * L, L)]
        tab[pl.ds(k * L, L)] = acc
        return carry

    lax.fori_loop(0, RPS // L, rbody, 0)
    pltpu.sync_copy(tab.at[pl.ds(0, RPS)],
                    out_hbm.at[c].at[pl.ds(s * RPS, RPS)])


_deg_call = functools.partial(
    pl.kernel,
    out_type=jax.ShapeDtypeStruct((NC, N_TAB), jnp.float32),
    mesh=_mesh,
    compiler_params=pltpu.CompilerParams(needs_layout_passes=False),
    scratch_types=[
        pltpu.VMEM((EPT,), jnp.int32),
        pltpu.VMEM((N_TAB,), jnp.float32),
        pltpu.VMEM((NS, RPS), jnp.float32),
        pltpu.VMEM_SHARED((NS, N_TAB), jnp.float32),
    ],
)(_deg_body)


FH = F // 2


def _gcn_body(tab_hbm, src2_hbm, dst2_hbm, zrow_hbm, out_hbm,
              srcv, dstv, buf0, buf1, g0, g1, s0, s1, tabsh, acc):
    c = lax.axis_index("c")
    s = lax.axis_index("s")
    w = s * NC + c
    pltpu.sync_copy(src2_hbm.at[w], srcv)
    pltpu.sync_copy(dst2_hbm.at[w], dstv)
    base = s * RPS

    for half in range(2):
        for t in range(5):
            rows = pl.ds(base + t * BB, BB)
            pltpu.sync_copy(tab_hbm.at[rows, pl.ds(half * FH, FH)], buf0)
            pltpu.sync_copy(buf0, tabsh.at[rows])
            pltpu.sync_copy(zrow_hbm, buf1)
            pltpu.sync_copy(buf1, acc.at[rows])
        plsc.subcore_barrier()

        pltpu.async_copy(tabsh.at[srcv.at[0]], buf0, g0)

        def body(b, carry):
            even = lax.rem(b, 2) == 0

            @pl.when(jnp.logical_and(even, b > 0))
            def _():
                pltpu.make_async_copy(buf1, acc.at[dstv.at[b]], s1).wait()

            @pl.when(jnp.logical_and(even, b + 1 < NB))
            def _():
                pltpu.async_copy(tabsh.at[srcv.at[b + 1]], buf1, g1)

            @pl.when(jnp.logical_not(even))
            def _():
                pltpu.make_async_copy(buf0, acc.at[dstv.at[b]], s0).wait()
                pltpu.async_copy(tabsh.at[srcv.at[b + 1]], buf0, g0)

            @pl.when(even)
            def _():
                pltpu.make_async_copy(tabsh.at[srcv.at[b]], buf0, g0).wait()
                pltpu.async_copy(buf0, acc.at[dstv.at[b]], s0, add=True)

            @pl.when(jnp.logical_not(even))
            def _():
                pltpu.make_async_copy(tabsh.at[srcv.at[b]], buf1, g1).wait()
                pltpu.async_copy(buf1, acc.at[dstv.at[b]], s1, add=True)

            return carry

        lax.fori_loop(0, NB, body, 0)
        pltpu.make_async_copy(buf0, acc.at[dstv.at[NB - 1]], s0).wait()
        plsc.subcore_barrier()
        for t in range(5):
            rows = pl.ds(base + t * BB, BB)
            pltpu.sync_copy(acc.at[rows], buf0)
            pltpu.sync_copy(buf0,
                            out_hbm.at[c].at[rows, pl.ds(half * FH, FH)])
        if half == 0:
            plsc.subcore_barrier()


_gcn_call = functools.partial(
    pl.kernel,
    out_type=jax.ShapeDtypeStruct((NC, N_TAB, F), jnp.float32),
    mesh=_mesh,
    compiler_params=pltpu.CompilerParams(
        needs_layout_passes=False, use_tc_tiling_on_sc=False),
    scratch_types=[
        pltpu.VMEM((NB, BB), jnp.int32),
        pltpu.VMEM((NB, BB), jnp.int32),
        pltpu.VMEM((BB, FH), jnp.float32),
        pltpu.VMEM((BB, FH), jnp.float32),
        pltpu.SemaphoreType.DMA,
        pltpu.SemaphoreType.DMA,
        pltpu.SemaphoreType.DMA,
        pltpu.SemaphoreType.DMA,
        pltpu.VMEM_SHARED((N_TAB, FH), jnp.float32),
        pltpu.VMEM_SHARED((N_TAB, FH), jnp.float32),
    ],
)(_gcn_body)


def _gat_body(gtab_hbm, asrc_hbm, adst_hbm, maxs_hbm,
              srcf_hbm, dstf_hbm, dst2_hbm, zeros_hbm, zrow_hbm,
              out_hbm, esum_hbm, ex_hbm,
              srcf, dstf, dstv, asv, adv, esv, exv, buf0, buf1,
              g0, g1, s0, s1, maxv, acc):
    c = lax.axis_index("c")
    s = lax.axis_index("s")
    w = s * NC + c
    pltpu.sync_copy(srcf_hbm.at[w], srcf)
    pltpu.sync_copy(dstf_hbm.at[w], dstf)
    pltpu.sync_copy(dst2_hbm.at[w], dstv)
    pltpu.sync_copy(asrc_hbm, asv)
    pltpu.sync_copy(adst_hbm, adv)
    pltpu.sync_copy(zeros_hbm, esv)
    pltpu.sync_copy(maxs_hbm, maxv)
    pltpu.sync_copy(zrow_hbm, buf0)
    base = s * RPS
    for t in range(5):
        pltpu.sync_copy(buf0, acc.at[pl.ds(base + t * BB, BB)])
    plsc.subcore_barrier()

    mx = maxv[...]

    def compute_ex(b):
        boff = b * BB
        for k in range(BB // L):
            off = boff + k * L
            sidx = srcf[pl.ds(off, L)]
            didx = dstf[pl.ds(off, L)]
            a_s = plsc.load_gather(asv, [sidx])
            a_d = plsc.load_gather(adv, [didx])
            e = a_s + a_d
            e = jnp.where(e >= 0.0, e, e * 0.2)
            m = mx + a_d
            m = jnp.where(m >= 0.0, m, m * 0.2)
            ex = jnp.exp(e - m)
            exv[pl.ds(off, L)] = ex
            plsc.addupdate_scatter(esv, [didx], ex)

    def scale_rows(bf, b):
        boff = b * BB

        def grp(gi, carry2):
            exk = exv[pl.ds(boff + gi * L, L)]
            for j in range(L):
                exb = _bcast(exk, j)
                r = gi * L + j
                for k in range(O // L):
                    bf[r, pl.ds(k * L, L)] = bf[r, pl.ds(k * L, L)] * exb
            return carry2

        lax.fori_loop(0, BB // L, grp, 0)

    pltpu.async_copy(gtab_hbm.at[srcf.at[pl.ds(0, BB)]], buf0, g0)

    def body(b, carry):
        even = lax.rem(b, 2) == 0
        boff = b * BB

        @pl.when(jnp.logical_and(even, b > 0))
        def _():
            pltpu.make_async_copy(buf1, acc.at[dstv.at[b]], s1).wait()

        @pl.when(jnp.logical_and(even, b + 1 < NB))
        def _():
            pltpu.async_copy(gtab_hbm.at[srcf.at[pl.ds(boff + BB, BB)]],
                             buf1, g1)

        @pl.when(jnp.logical_not(even))
        def _():
            pltpu.make_async_copy(buf0, acc.at[dstv.at[b]], s0).wait()
            pltpu.async_copy(gtab_hbm.at[srcf.at[pl.ds(boff + BB, BB)]],
                             buf0, g0)

        compute_ex(b)

        @pl.when(even)
        def _():
            pltpu.make_async_copy(gtab_hbm.at[srcf.at[pl.ds(boff, BB)]],
                                  buf0, g0).wait()
            scale_rows(buf0, b)
            pltpu.async_copy(buf0, acc.at[dstv.at[b]], s0, add=True)

        @pl.when(jnp.logical_not(even))
        def _():
            pltpu.make_async_copy(gtab_hbm.at[srcf.at[pl.ds(boff, BB)]],
                                  buf1, g1).wait()
            scale_rows(buf1, b)
            pltpu.async_copy(buf1, acc.at[dstv.at[b]], s1, add=True)

        return carry

    lax.fori_loop(0, NB, body, 0)
    pltpu.make_async_copy(buf0, acc.at[dstv.at[NB - 1]], s0).wait()
    plsc.subcore_barrier()
    pltpu.sync_copy(esv, esum_hbm.at[w])
    pltpu.sync_copy(exv, ex_hbm.at[w])
    for t in range(5):
        pltpu.sync_copy(acc.at[pl.ds(base + t * BB, BB)], buf0)
        pltpu.sync_copy(buf0, out_hbm.at[c].at[pl.ds(base + t * BB, BB)])


_gat_call = functools.partial(
    pl.kernel,
    out_type=(
        jax.ShapeDtypeStruct((NC, N_TAB, O), jnp.float32),
        jax.ShapeDtypeStruct((NW, N_TAB), jnp.float32),
        jax.ShapeDtypeStruct((NW, EPT), jnp.float32),
    ),
    mesh=_mesh,
    compiler_params=pltpu.CompilerParams(
        needs_layout_passes=False, use_tc_tiling_on_sc=False),
    scratch_types=[
        pltpu.VMEM((EPT,), jnp.int32),
        pltpu.VMEM((EPT,), jnp.int32),
        pltpu.VMEM((NB, BB), jnp.int32),
        pltpu.VMEM((N_TAB,), jnp.float32),
        pltpu.VMEM((N_TAB,), jnp.float32),
        pltpu.VMEM((N_TAB,), jnp.float32),
        pltpu.VMEM((EPT,), jnp.float32),
        pltpu.VMEM((BB, O), jnp.float32),
        pltpu.VMEM((BB, O), jnp.float32),
        pltpu.SemaphoreType.DMA,
        pltpu.SemaphoreType.DMA,
        pltpu.SemaphoreType.DMA,
        pltpu.SemaphoreType.DMA,
        pltpu.VMEM((L,), jnp.float32),
        pltpu.VMEM_SHARED((N_TAB, O), jnp.float32),
    ],
)(_gat_body)


def _alpha_body(ex_hbm, dstf_hbm, rec_hbm, out_hbm, exv, dstf, recv, av):
    w = lax.axis_index("s") * NC + lax.axis_index("c")
    pltpu.sync_copy(ex_hbm.at[w], exv)
    pltpu.sync_copy(dstf_hbm.at[w], dstf)
    pltpu.sync_copy(rec_hbm, recv)

    def body(i, carry):
        off = i * L
        didx = dstf[pl.ds(off, L)]
        r = plsc.load_gather(recv, [didx])
        av[pl.ds(off, L)] = exv[pl.ds(off, L)] * r
        return carry

    lax.fori_loop(0, EPT // L, body, 0)
    pltpu.sync_copy(av, out_hbm.at[w])


_alpha_call = functools.partial(
    pl.kernel,
    out_type=jax.ShapeDtypeStruct((NW, EPT), jnp.float32),
    mesh=_mesh,
    compiler_params=pltpu.CompilerParams(needs_layout_passes=False),
    scratch_types=[
        pltpu.VMEM((EPT,), jnp.float32),
        pltpu.VMEM((EPT,), jnp.int32),
        pltpu.VMEM((N_TAB,), jnp.float32),
        pltpu.VMEM((EPT,), jnp.float32),
    ],
)(_alpha_body)


def _t1_body(x_ref, w1_ref, degp_ref, h0p_ref, dinv_ref):
    deg = degp_ref[0, :] + degp_ref[1, :] + 1.0
    dinv = lax.rsqrt(deg)[:, None]
    h0 = jnp.dot(x_ref[...], w1_ref[...], preferred_element_type=jnp.float32)
    h0p_ref[pl.ds(0, N), :] = h0 * dinv[:N]
    h0p_ref[pl.ds(N, N_TAB - N), :] = jnp.zeros((N_TAB - N, F), jnp.float32)
    dinv_ref[...] = dinv


def _t1(x, W1, degp):
    return pl.pallas_call(
        _t1_body,
        out_shape=[
            jax.ShapeDtypeStruct((N_TAB, F), jnp.float32),
            jax.ShapeDtypeStruct((N_TAB, 1), jnp.float32),
        ],
    )(x, W1, degp)


def _t2_body(p_ref, h0p_ref, dinv_ref, b1_ref, w2_ref, as_ref,
             ad_ref, g_ref, asrc_ref, adst_ref, exs_ref, maxs_ref):
    S = p_ref[0] + p_ref[1] + h0p_ref[...]
    h = jnp.maximum(dinv_ref[...] * S + b1_ref[...], 0.0)
    g = jnp.dot(h, w2_ref[...], preferred_element_type=jnp.float32)
    asrc = jnp.dot(g, as_ref[...], preferred_element_type=jnp.float32)
    adst = jnp.dot(g, ad_ref[...], preferred_element_type=jnp.float32)
    maxs = jnp.max(asrc)
    m = maxs + adst
    m = jnp.where(m >= 0.0, m, m * 0.2)
    e = asrc + adst
    e = jnp.where(e >= 0.0, e, e * 0.2)
    g_ref[...] = g
    asrc_ref[...] = asrc[:, 0]
    adst_ref[...] = adst[:, 0]
    exs_ref[...] = jnp.exp(e - m)
    maxs_ref[...] = jnp.full((1, L), maxs, jnp.float32)


def _t2(p, h0p, dinv, b1, W2, att_src, att_dst):
    return pl.pallas_call(
        _t2_body,
        out_shape=[
            jax.ShapeDtypeStruct((N_TAB, O), jnp.float32),
            jax.ShapeDtypeStruct((N_TAB,), jnp.float32),
            jax.ShapeDtypeStruct((N_TAB,), jnp.float32),
            jax.ShapeDtypeStruct((N_TAB, 1), jnp.float32),
            jax.ShapeDtypeStruct((1, L), jnp.float32),
        ],
    )(p, h0p, dinv, b1, W2, att_src, att_dst)


def _t3_body(pg_ref, esump_ref, exs_ref, g_ref, b2_ref,
             out_ref, aself_ref, rec_ref):
    exs = exs_ref[...]
    esum = jnp.sum(esump_ref[...], axis=0)[:, None] + exs
    rec = 1.0 / (esum + 1e-16)
    g = g_ref[...]
    full = (pg_ref[0] + pg_ref[1] + exs * g) * rec + b2_ref[...]
    out_ref[...] = full[:N]
    aself_ref[...] = (exs * rec)[:N, 0]
    rec_ref[...] = rec[:, 0]


def _t3(pg, esump, exs, g, b2):
    return pl.pallas_call(
        _t3_body,
        out_shape=[
            jax.ShapeDtypeStruct((N, O), jnp.float32),
            jax.ShapeDtypeStruct((N,), jnp.float32),
            jax.ShapeDtypeStruct((N_TAB,), jnp.float32),
        ],
    )(pg, esump, exs, g, b2)


def kernel(x, edge_index, W1, b1, W2, att_src, att_dst, b2):
    src = edge_index[0]
    dst = edge_index[1]
    pad = jnp.full((E_PAD - E,), N, jnp.int32)
    pad_f = jnp.full((E_PAD - E,), N + 1, jnp.int32)
    srcp = jnp.concatenate([src, pad])
    dstp = jnp.concatenate([dst, pad])
    srcf = jnp.concatenate([src, pad_f]).reshape(NW, EPT)
    dstf = jnp.concatenate([dst, pad_f]).reshape(NW, EPT)
    src2 = srcp.reshape(NW, NB, BB)
    dst2 = dstp.reshape(NW, NB, BB)
    zeros_nt = jnp.zeros((N_TAB,), jnp.float32)
    zrow_f = jnp.zeros((BB, FH), jnp.float32)
    zrow_o = jnp.zeros((BB, O), jnp.float32)

    degp = _deg_call(dstf, zeros_nt)
    h0p, dinv = _t1(x, W1, degp)
    P = _gcn_call(h0p, src2, dst2, zrow_f)
    g, asrc, adst, exs, maxs = _t2(P, h0p, dinv, b1.reshape(1, F), W2,
                                   att_src.reshape(O, 1), att_dst.reshape(O, 1))
    Pg, esump, exe = _gat_call(
        g, asrc, adst, maxs.reshape(L),
        srcf, dstf, dst2, zeros_nt, zrow_o)
    out, aself, rec = _t3(Pg, esump, exs, g, b2.reshape(1, O))
    alpha_e = _alpha_call(exe, dstf, rec).reshape(E_PAD)[:E]
    alpha = jnp.concatenate([alpha_e, aself])
    ar = jnp.arange(N, dtype=edge_index.dtype)
    ei_full = jnp.stack([jnp.concatenate([src, ar]), jnp.concatenate([dst, ar])])
    return (out, (ei_full, alpha))


# speedup vs baseline: 1.0961x; 1.0164x over previous
import functools

import jax
import jax.numpy as jnp
from jax import lax
from jax.experimental import pallas as pl
from jax.experimental.pallas import tpu as pltpu
from jax.experimental.pallas import tpu_sc as plsc

N = 10000
E = 320000
F = 128
O = 64
NC = 2
NS = 16
NW = NC * NS
L = 16

BB = 128
NB = 79
EPT = NB * BB
E_PAD = NW * EPT
N_TAB = 10240
RPS = N_TAB // NS

_mesh = plsc.VectorSubcoreMesh(
    core_axis_name="c", subcore_axis_name="s", num_cores=NC, num_subcores=NS)


def _bcast(v, j):
    return lax.gather(
        v, jnp.full((L, 1), j, jnp.int32),
        lax.GatherDimensionNumbers(
            offset_dims=(), collapsed_slice_dims=(0,), start_index_map=(0,)),
        (1,), mode=lax.GatherScatterMode.PROMISE_IN_BOUNDS)


def _deg_body(dstf_hbm, zeros_hbm, out_hbm, dstf, tab, red, stage):
    c = lax.axis_index("c")
    s = lax.axis_index("s")
    w = s * NC + c
    pltpu.sync_copy(dstf_hbm.at[w], dstf)
    pltpu.sync_copy(zeros_hbm, tab)
    ones = jnp.full((L,), 1.0, jnp.float32)

    def body(i, carry):
        idx = dstf[pl.ds(i * L, L)]
        plsc.addupdate_scatter(tab, [idx], ones)
        return carry

    lax.fori_loop(0, EPT // L, body, 0)
    pltpu.sync_copy(tab, stage.at[s])
    plsc.subcore_barrier()
    pltpu.sync_copy(stage.at[:, pl.ds(s * RPS, RPS)], red)

    def rbody(k, carry):
        acc = jnp.zeros((L,), jnp.float32)
        for t in range(NS):
            acc = acc + red[t, pl.ds(k * L, L)]
        tab[pl.ds(k * L, L)] = acc
        return carry

    lax.fori_loop(0, RPS // L, rbody, 0)
    pltpu.sync_copy(tab.at[pl.ds(0, RPS)],
                    out_hbm.at[c].at[pl.ds(s * RPS, RPS)])


_deg_call = functools.partial(
    pl.kernel,
    out_type=jax.ShapeDtypeStruct((NC, N_TAB), jnp.float32),
    mesh=_mesh,
    compiler_params=pltpu.CompilerParams(needs_layout_passes=False),
    scratch_types=[
        pltpu.VMEM((EPT,), jnp.int32),
        pltpu.VMEM((N_TAB,), jnp.float32),
        pltpu.VMEM((NS, RPS), jnp.float32),
        pltpu.VMEM_SHARED((NS, N_TAB), jnp.float32),
    ],
)(_deg_body)


FH = F // 2


def _gcn_body(tab_hbm, src2_hbm, dst2_hbm, zrow_hbm, out_hbm,
              srcv, dstv, buf0, buf1, g0, g1, s0, s1, tabsh, acc):
    c = lax.axis_index("c")
    s = lax.axis_index("s")
    w = s * NC + c
    pltpu.sync_copy(src2_hbm.at[w], srcv)
    pltpu.sync_copy(dst2_hbm.at[w], dstv)
    base = s * RPS

    for half in range(2):
        for t in range(5):
            rows = pl.ds(base + t * BB, BB)
            pltpu.sync_copy(tab_hbm.at[rows, pl.ds(half * FH, FH)], buf0)
            pltpu.sync_copy(buf0, tabsh.at[rows])
            pltpu.sync_copy(zrow_hbm, buf1)
            pltpu.sync_copy(buf1, acc.at[rows])
        plsc.subcore_barrier()

        pltpu.async_copy(tabsh.at[srcv.at[0]], buf0, g0)

        def body(b, carry):
            even = lax.rem(b, 2) == 0

            @pl.when(jnp.logical_and(even, b > 0))
            def _():
                pltpu.make_async_copy(buf1, acc.at[dstv.at[b]], s1).wait()

            @pl.when(jnp.logical_and(even, b + 1 < NB))
            def _():
                pltpu.async_copy(tabsh.at[srcv.at[b + 1]], buf1, g1)

            @pl.when(jnp.logical_not(even))
            def _():
                pltpu.make_async_copy(buf0, acc.at[dstv.at[b]], s0).wait()
                pltpu.async_copy(tabsh.at[srcv.at[b + 1]], buf0, g0)

            @pl.when(even)
            def _():
                pltpu.make_async_copy(tabsh.at[srcv.at[b]], buf0, g0).wait()
                pltpu.async_copy(buf0, acc.at[dstv.at[b]], s0, add=True)

            @pl.when(jnp.logical_not(even))
            def _():
                pltpu.make_async_copy(tabsh.at[srcv.at[b]], buf1, g1).wait()
                pltpu.async_copy(buf1, acc.at[dstv.at[b]], s1, add=True)

            return carry

        lax.fori_loop(0, NB, body, 0)
        pltpu.make_async_copy(buf0, acc.at[dstv.at[NB - 1]], s0).wait()
        plsc.subcore_barrier()
        for t in range(5):
            rows = pl.ds(base + t * BB, BB)
            pltpu.sync_copy(acc.at[rows], buf0)
            pltpu.sync_copy(buf0,
                            out_hbm.at[c].at[rows, pl.ds(half * FH, FH)])
        if half == 0:
            plsc.subcore_barrier()


_gcn_call = functools.partial(
    pl.kernel,
    out_type=jax.ShapeDtypeStruct((NC, N_TAB, F), jnp.float32),
    mesh=_mesh,
    compiler_params=pltpu.CompilerParams(
        needs_layout_passes=False, use_tc_tiling_on_sc=False),
    scratch_types=[
        pltpu.VMEM((NB, BB), jnp.int32),
        pltpu.VMEM((NB, BB), jnp.int32),
        pltpu.VMEM((BB, FH), jnp.float32),
        pltpu.VMEM((BB, FH), jnp.float32),
        pltpu.SemaphoreType.DMA,
        pltpu.SemaphoreType.DMA,
        pltpu.SemaphoreType.DMA,
        pltpu.SemaphoreType.DMA,
        pltpu.VMEM_SHARED((N_TAB, FH), jnp.float32),
        pltpu.VMEM_SHARED((N_TAB, FH), jnp.float32),
    ],
)(_gcn_body)


def _gat_body(gtab_hbm, asrc_hbm, adst_hbm, maxs_hbm,
              srcf_hbm, dstf_hbm, dst2_hbm, zeros_hbm, zrow_hbm,
              out_hbm, esum_hbm, ex_hbm,
              srcf, dstf, dstv, asv, adv, esv, exv, buf0, buf1,
              g0, g1, s0, s1, maxv, acc):
    c = lax.axis_index("c")
    s = lax.axis_index("s")
    w = s * NC + c
    pltpu.sync_copy(srcf_hbm.at[w], srcf)
    pltpu.sync_copy(dstf_hbm.at[w], dstf)
    pltpu.sync_copy(dst2_hbm.at[w], dstv)
    pltpu.sync_copy(asrc_hbm, asv)
    pltpu.sync_copy(adst_hbm, adv)
    pltpu.sync_copy(zeros_hbm, esv)
    pltpu.sync_copy(maxs_hbm, maxv)
    pltpu.sync_copy(zrow_hbm, buf0)
    base = s * RPS
    for t in range(5):
        pltpu.sync_copy(buf0, acc.at[pl.ds(base + t * BB, BB)])
    plsc.subcore_barrier()

    mx = maxv[...]

    def compute_ex(b):
        boff = b * BB
        for k in range(BB // L):
            off = boff + k * L
            sidx = srcf[pl.ds(off, L)]
            didx = dstf[pl.ds(off, L)]
            a_s = plsc.load_gather(asv, [sidx])
            a_d = plsc.load_gather(adv, [didx])
            e = a_s + a_d
            e = jnp.where(e >= 0.0, e, e * 0.2)
            m = mx + a_d
            m = jnp.where(m >= 0.0, m, m * 0.2)
            ex = jnp.exp(e - m)
            exv[pl.ds(off, L)] = ex
            plsc.addupdate_scatter(esv, [didx], ex)

    def scale_rows(bf, b):
        boff = b * BB

        def grp(gi, carry2):
            exk = exv[pl.ds(boff + gi * L, L)]
            for j in range(L):
                exb = _bcast(exk, j)
                r = gi * L + j
                for k in range(O // L):
                    bf[r, pl.ds(k * L, L)] = bf[r, pl.ds(k * L, L)] * exb
            return carry2

        lax.fori_loop(0, BB // L, grp, 0)

    pltpu.async_copy(gtab_hbm.at[srcf.at[pl.ds(0, BB)]], buf0, g0)

    def body(b, carry):
        even = lax.rem(b, 2) == 0
        boff = b * BB

        compute_ex(b)

        @pl.when(jnp.logical_and(even, b > 0))
        def _():
            pltpu.make_async_copy(buf1, acc.at[dstv.at[b]], s1).wait()

        @pl.when(jnp.logical_and(even, b + 1 < NB))
        def _():
            pltpu.async_copy(gtab_hbm.at[srcf.at[pl.ds(boff + BB, BB)]],
                             buf1, g1)

        @pl.when(jnp.logical_not(even))
        def _():
            pltpu.make_async_copy(buf0, acc.at[dstv.at[b]], s0).wait()
            pltpu.async_copy(gtab_hbm.at[srcf.at[pl.ds(boff + BB, BB)]],
                             buf0, g0)

        @pl.when(even)
        def _():
            pltpu.make_async_copy(gtab_hbm.at[srcf.at[pl.ds(boff, BB)]],
                                  buf0, g0).wait()
            scale_rows(buf0, b)
            pltpu.async_copy(buf0, acc.at[dstv.at[b]], s0, add=True)

        @pl.when(jnp.logical_not(even))
        def _():
            pltpu.make_async_copy(gtab_hbm.at[srcf.at[pl.ds(boff, BB)]],
                                  buf1, g1).wait()
            scale_rows(buf1, b)
            pltpu.async_copy(buf1, acc.at[dstv.at[b]], s1, add=True)

        return carry

    lax.fori_loop(0, NB, body, 0)
    pltpu.make_async_copy(buf0, acc.at[dstv.at[NB - 1]], s0).wait()
    plsc.subcore_barrier()
    pltpu.sync_copy(esv, esum_hbm.at[w])
    pltpu.sync_copy(exv, ex_hbm.at[w])
    for t in range(5):
        pltpu.sync_copy(acc.at[pl.ds(base + t * BB, BB)], buf0)
        pltpu.sync_copy(buf0, out_hbm.at[c].at[pl.ds(base + t * BB, BB)])


_gat_call = functools.partial(
    pl.kernel,
    out_type=(
        jax.ShapeDtypeStruct((NC, N_TAB, O), jnp.float32),
        jax.ShapeDtypeStruct((NW, N_TAB), jnp.float32),
        jax.ShapeDtypeStruct((NW, EPT), jnp.float32),
    ),
    mesh=_mesh,
    compiler_params=pltpu.CompilerParams(
        needs_layout_passes=False, use_tc_tiling_on_sc=False),
    scratch_types=[
        pltpu.VMEM((EPT,), jnp.int32),
        pltpu.VMEM((EPT,), jnp.int32),
        pltpu.VMEM((NB, BB), jnp.int32),
        pltpu.VMEM((N_TAB,), jnp.float32),
        pltpu.VMEM((N_TAB,), jnp.float32),
        pltpu.VMEM((N_TAB,), jnp.float32),
        pltpu.VMEM((EPT,), jnp.float32),
        pltpu.VMEM((BB, O), jnp.float32),
        pltpu.VMEM((BB, O), jnp.float32),
        pltpu.SemaphoreType.DMA,
        pltpu.SemaphoreType.DMA,
        pltpu.SemaphoreType.DMA,
        pltpu.SemaphoreType.DMA,
        pltpu.VMEM((L,), jnp.float32),
        pltpu.VMEM_SHARED((N_TAB, O), jnp.float32),
    ],
)(_gat_body)


def _alpha_body(ex_hbm, dstf_hbm, rec_hbm, out_hbm, exv, dstf, recv, av):
    w = lax.axis_index("s") * NC + lax.axis_index("c")
    pltpu.sync_copy(ex_hbm.at[w], exv)
    pltpu.sync_copy(dstf_hbm.at[w], dstf)
    pltpu.sync_copy(rec_hbm, recv)

    def body(i, carry):
        off = i * L
        didx = dstf[pl.ds(off, L)]
        r = plsc.load_gather(recv, [didx])
        av[pl.ds(off, L)] = exv[pl.ds(off, L)] * r
        return carry

    lax.fori_loop(0, EPT // L, body, 0)
    pltpu.sync_copy(av, out_hbm.at[w])


_alpha_call = functools.partial(
    pl.kernel,
    out_type=jax.ShapeDtypeStruct((NW, EPT), jnp.float32),
    mesh=_mesh,
    compiler_params=pltpu.CompilerParams(needs_layout_passes=False),
    scratch_types=[
        pltpu.VMEM((EPT,), jnp.float32),
        pltpu.VMEM((EPT,), jnp.int32),
        pltpu.VMEM((N_TAB,), jnp.float32),
        pltpu.VMEM((EPT,), jnp.float32),
    ],
)(_alpha_body)


def _t1_body(x_ref, w1_ref, degp_ref, h0p_ref, dinv_ref):
    deg = degp_ref[0, :] + degp_ref[1, :] + 1.0
    dinv = lax.rsqrt(deg)[:, None]
    h0 = jnp.dot(x_ref[...], w1_ref[...], preferred_element_type=jnp.float32)
    h0p_ref[pl.ds(0, N), :] = h0 * dinv[:N]
    h0p_ref[pl.ds(N, N_TAB - N), :] = jnp.zeros((N_TAB - N, F), jnp.float32)
    dinv_ref[...] = dinv


def _t1(x, W1, degp):
    return pl.pallas_call(
        _t1_body,
        out_shape=[
            jax.ShapeDtypeStruct((N_TAB, F), jnp.float32),
            jax.ShapeDtypeStruct((N_TAB, 1), jnp.float32),
        ],
    )(x, W1, degp)


def _t2_body(p_ref, h0p_ref, dinv_ref, b1_ref, w2_ref, as_ref,
             ad_ref, g_ref, asrc_ref, adst_ref, exs_ref, maxs_ref):
    S = p_ref[0] + p_ref[1] + h0p_ref[...]
    h = jnp.maximum(dinv_ref[...] * S + b1_ref[...], 0.0)
    g = jnp.dot(h, w2_ref[...], preferred_element_type=jnp.float32)
    asrc = jnp.dot(g, as_ref[...], preferred_element_type=jnp.float32)
    adst = jnp.dot(g, ad_ref[...], preferred_element_type=jnp.float32)
    maxs = jnp.max(asrc)
    m = maxs + adst
    m = jnp.where(m >= 0.0, m, m * 0.2)
    e = asrc + adst
    e = jnp.where(e >= 0.0, e, e * 0.2)
    g_ref[...] = g
    asrc_ref[...] = asrc[:, 0]
    adst_ref[...] = adst[:, 0]
    exs_ref[...] = jnp.exp(e - m)
    maxs_ref[...] = jnp.full((1, L), maxs, jnp.float32)


def _t2(p, h0p, dinv, b1, W2, att_src, att_dst):
    return pl.pallas_call(
        _t2_body,
        out_shape=[
            jax.ShapeDtypeStruct((N_TAB, O), jnp.float32),
            jax.ShapeDtypeStruct((N_TAB,), jnp.float32),
            jax.ShapeDtypeStruct((N_TAB,), jnp.float32),
            jax.ShapeDtypeStruct((N_TAB, 1), jnp.float32),
            jax.ShapeDtypeStruct((1, L), jnp.float32),
        ],
    )(p, h0p, dinv, b1, W2, att_src, att_dst)


def _t3_body(pg_ref, esump_ref, exs_ref, g_ref, b2_ref,
             out_ref, aself_ref, rec_ref):
    exs = exs_ref[...]
    esum = jnp.sum(esump_ref[...], axis=0)[:, None] + exs
    rec = 1.0 / (esum + 1e-16)
    g = g_ref[...]
    full = (pg_ref[0] + pg_ref[1] + exs * g) * rec + b2_ref[...]
    out_ref[...] = full[:N]
    aself_ref[...] = (exs * rec)[:N, 0]
    rec_ref[...] = rec[:, 0]


def _t3(pg, esump, exs, g, b2):
    return pl.pallas_call(
        _t3_body,
        out_shape=[
            jax.ShapeDtypeStruct((N, O), jnp.float32),
            jax.ShapeDtypeStruct((N,), jnp.float32),
            jax.ShapeDtypeStruct((N_TAB,), jnp.float32),
        ],
    )(pg, esump, exs, g, b2)


def kernel(x, edge_index, W1, b1, W2, att_src, att_dst, b2):
    src = edge_index[0]
    dst = edge_index[1]
    pad = jnp.full((E_PAD - E,), N, jnp.int32)
    pad_f = jnp.full((E_PAD - E,), N + 1, jnp.int32)
    srcp = jnp.concatenate([src, pad])
    dstp = jnp.concatenate([dst, pad])
    srcf = jnp.concatenate([src, pad_f]).reshape(NW, EPT)
    dstf = jnp.concatenate([dst, pad_f]).reshape(NW, EPT)
    src2 = srcp.reshape(NW, NB, BB)
    dst2 = dstp.reshape(NW, NB, BB)
    zeros_nt = jnp.zeros((N_TAB,), jnp.float32)
    zrow_f = jnp.zeros((BB, FH), jnp.float32)
    zrow_o = jnp.zeros((BB, O), jnp.float32)

    degp = _deg_call(dstf, zeros_nt)
    h0p, dinv = _t1(x, W1, degp)
    P = _gcn_call(h0p, src2, dst2, zrow_f)
    g, asrc, adst, exs, maxs = _t2(P, h0p, dinv, b1.reshape(1, F), W2,
                                   att_src.reshape(O, 1), att_dst.reshape(O, 1))
    Pg, esump, exe = _gat_call(
        g, asrc, adst, maxs.reshape(L),
        srcf, dstf, dst2, zeros_nt, zrow_o)
    out, aself, rec = _t3(Pg, esump, exs, g, b2.reshape(1, O))
    alpha_e = _alpha_call(exe, dstf, rec).reshape(E_PAD)[:E]
    alpha = jnp.concatenate([alpha_e, aself])
    ar = jnp.arange(N, dtype=edge_index.dtype)
    ei_full = jnp.stack([jnp.concatenate([src, ar]), jnp.concatenate([dst, ar])])
    return (out, (ei_full, alpha))


# speedup vs baseline: 1.1125x; 1.0150x over previous
import functools

import jax
import jax.numpy as jnp
from jax import lax
from jax.experimental import pallas as pl
from jax.experimental.pallas import tpu as pltpu
from jax.experimental.pallas import tpu_sc as plsc

N = 10000
E = 320000
F = 128
O = 64
NC = 2
NS = 16
NW = NC * NS
L = 16

BB = 128
NB = 79
EPT = NB * BB
E_PAD = NW * EPT
N_TAB = 10240
RPS = N_TAB // NS

_mesh = plsc.VectorSubcoreMesh(
    core_axis_name="c", subcore_axis_name="s", num_cores=NC, num_subcores=NS)


def _bcast(v, j):
    return lax.gather(
        v, jnp.full((L, 1), j, jnp.int32),
        lax.GatherDimensionNumbers(
            offset_dims=(), collapsed_slice_dims=(0,), start_index_map=(0,)),
        (1,), mode=lax.GatherScatterMode.PROMISE_IN_BOUNDS)


def _deg_body(dstf_hbm, zeros_hbm, out_hbm, dstf, tab, red, stage):
    c = lax.axis_index("c")
    s = lax.axis_index("s")
    w = s * NC + c
    pltpu.sync_copy(dstf_hbm.at[w], dstf)
    pltpu.sync_copy(zeros_hbm, tab)
    ones = jnp.full((L,), 1.0, jnp.float32)

    def body(i, carry):
        idx = dstf[pl.ds(i * L, L)]
        plsc.addupdate_scatter(tab, [idx], ones)
        return carry

    lax.fori_loop(0, EPT // L, body, 0)
    pltpu.sync_copy(tab, stage.at[s])
    plsc.subcore_barrier()
    pltpu.sync_copy(stage.at[:, pl.ds(s * RPS, RPS)], red)

    def rbody(k, carry):
        acc = jnp.zeros((L,), jnp.float32)
        for t in range(NS):
            acc = acc + red[t, pl.ds(k * L, L)]
        tab[pl.ds(k * L, L)] = acc
        return carry

    lax.fori_loop(0, RPS // L, rbody, 0)
    pltpu.sync_copy(tab.at[pl.ds(0, RPS)],
                    out_hbm.at[c].at[pl.ds(s * RPS, RPS)])


_deg_call = functools.partial(
    pl.kernel,
    out_type=jax.ShapeDtypeStruct((NC, N_TAB), jnp.float32),
    mesh=_mesh,
    compiler_params=pltpu.CompilerParams(needs_layout_passes=False),
    scratch_types=[
        pltpu.VMEM((EPT,), jnp.int32),
        pltpu.VMEM((N_TAB,), jnp.float32),
        pltpu.VMEM((NS, RPS), jnp.float32),
        pltpu.VMEM_SHARED((NS, N_TAB), jnp.float32),
    ],
)(_deg_body)


FH = F // 2


def _gcn_body(tab_hbm, src2_hbm, dst2_hbm, zrow_hbm, out_hbm,
              srcv, dstv, buf0, buf1, g0, g1, s0, s1, tabsh, acc):
    c = lax.axis_index("c")
    s = lax.axis_index("s")
    w = s * NC + c
    pltpu.sync_copy(src2_hbm.at[w], srcv)
    pltpu.sync_copy(dst2_hbm.at[w], dstv)
    base = s * RPS

    for half in range(2):
        pltpu.sync_copy(zrow_hbm, buf1)
        for t in range(5):
            rows = pl.ds(base + t * BB, BB)
            pltpu.sync_copy(tab_hbm.at[rows, pl.ds(half * FH, FH)], buf0)
            pltpu.sync_copy(buf0, tabsh.at[rows])
            pltpu.sync_copy(buf1, acc.at[rows])
        plsc.subcore_barrier()

        pltpu.async_copy(tabsh.at[srcv.at[0]], buf0, g0)

        def body(b, carry):
            even = lax.rem(b, 2) == 0

            @pl.when(jnp.logical_and(even, b > 0))
            def _():
                pltpu.make_async_copy(buf1, acc.at[dstv.at[b]], s1).wait()

            @pl.when(jnp.logical_and(even, b + 1 < NB))
            def _():
                pltpu.async_copy(tabsh.at[srcv.at[b + 1]], buf1, g1)

            @pl.when(jnp.logical_not(even))
            def _():
                pltpu.make_async_copy(buf0, acc.at[dstv.at[b]], s0).wait()
                pltpu.async_copy(tabsh.at[srcv.at[b + 1]], buf0, g0)

            @pl.when(even)
            def _():
                pltpu.make_async_copy(tabsh.at[srcv.at[b]], buf0, g0).wait()
                pltpu.async_copy(buf0, acc.at[dstv.at[b]], s0, add=True)

            @pl.when(jnp.logical_not(even))
            def _():
                pltpu.make_async_copy(tabsh.at[srcv.at[b]], buf1, g1).wait()
                pltpu.async_copy(buf1, acc.at[dstv.at[b]], s1, add=True)

            return carry

        lax.fori_loop(0, NB, body, 0)
        pltpu.make_async_copy(buf0, acc.at[dstv.at[NB - 1]], s0).wait()
        plsc.subcore_barrier()
        for t in range(5):
            rows = pl.ds(base + t * BB, BB)
            pltpu.sync_copy(acc.at[rows], buf0)
            pltpu.sync_copy(buf0,
                            out_hbm.at[c].at[rows, pl.ds(half * FH, FH)])
        if half == 0:
            plsc.subcore_barrier()


_gcn_call = functools.partial(
    pl.kernel,
    out_type=jax.ShapeDtypeStruct((NC, N_TAB, F), jnp.float32),
    mesh=_mesh,
    compiler_params=pltpu.CompilerParams(
        needs_layout_passes=False, use_tc_tiling_on_sc=False),
    scratch_types=[
        pltpu.VMEM((NB, BB), jnp.int32),
        pltpu.VMEM((NB, BB), jnp.int32),
        pltpu.VMEM((BB, FH), jnp.float32),
        pltpu.VMEM((BB, FH), jnp.float32),
        pltpu.SemaphoreType.DMA,
        pltpu.SemaphoreType.DMA,
        pltpu.SemaphoreType.DMA,
        pltpu.SemaphoreType.DMA,
        pltpu.VMEM_SHARED((N_TAB, FH), jnp.float32),
        pltpu.VMEM_SHARED((N_TAB, FH), jnp.float32),
    ],
)(_gcn_body)


def _gat_body(gtab_hbm, asrc_hbm, adst_hbm, maxs_hbm,
              srcf_hbm, dstf_hbm, dst2_hbm, zeros_hbm, zrow_hbm,
              out_hbm, esum_hbm, ex_hbm,
              srcf, dstf, dstv, asv, adv, esv, exv, buf0, buf1,
              g0, g1, s0, s1, maxv, acc):
    c = lax.axis_index("c")
    s = lax.axis_index("s")
    w = s * NC + c
    pltpu.sync_copy(srcf_hbm.at[w], srcf)
    pltpu.sync_copy(dstf_hbm.at[w], dstf)
    pltpu.sync_copy(dst2_hbm.at[w], dstv)
    pltpu.sync_copy(asrc_hbm, asv)
    pltpu.sync_copy(adst_hbm, adv)
    pltpu.sync_copy(zeros_hbm, esv)
    pltpu.sync_copy(maxs_hbm, maxv)
    pltpu.sync_copy(zrow_hbm, buf0)
    base = s * RPS
    for t in range(5):
        pltpu.sync_copy(buf0, acc.at[pl.ds(base + t * BB, BB)])
    plsc.subcore_barrier()

    mx = maxv[...]

    def compute_ex(b):
        boff = b * BB
        for k in range(BB // L):
            off = boff + k * L
            sidx = srcf[pl.ds(off, L)]
            didx = dstf[pl.ds(off, L)]
            a_s = plsc.load_gather(asv, [sidx])
            a_d = plsc.load_gather(adv, [didx])
            e = a_s + a_d
            e = jnp.where(e >= 0.0, e, e * 0.2)
            m = mx + a_d
            m = jnp.where(m >= 0.0, m, m * 0.2)
            ex = jnp.exp(e - m)
            exv[pl.ds(off, L)] = ex
            plsc.addupdate_scatter(esv, [didx], ex)

    def scale_rows(bf, b):
        boff = b * BB

        def grp(gi, carry2):
            exk = exv[pl.ds(boff + gi * L, L)]
            for j in range(L):
                exb = _bcast(exk, j)
                r = gi * L + j
                for k in range(O // L):
                    bf[r, pl.ds(k * L, L)] = bf[r, pl.ds(k * L, L)] * exb
            return carry2

        lax.fori_loop(0, BB // L, grp, 0)

    pltpu.async_copy(gtab_hbm.at[srcf.at[pl.ds(0, BB)]], buf0, g0)

    def body(b, carry):
        even = lax.rem(b, 2) == 0
        boff = b * BB

        compute_ex(b)

        @pl.when(jnp.logical_and(even, b > 0))
        def _():
            pltpu.make_async_copy(buf1, acc.at[dstv.at[b]], s1).wait()

        @pl.when(jnp.logical_and(even, b + 1 < NB))
        def _():
            pltpu.async_copy(gtab_hbm.at[srcf.at[pl.ds(boff + BB, BB)]],
                             buf1, g1)

        @pl.when(jnp.logical_not(even))
        def _():
            pltpu.make_async_copy(buf0, acc.at[dstv.at[b]], s0).wait()
            pltpu.async_copy(gtab_hbm.at[srcf.at[pl.ds(boff + BB, BB)]],
                             buf0, g0)

        @pl.when(even)
        def _():
            pltpu.make_async_copy(gtab_hbm.at[srcf.at[pl.ds(boff, BB)]],
                                  buf0, g0).wait()
            scale_rows(buf0, b)
            pltpu.async_copy(buf0, acc.at[dstv.at[b]], s0, add=True)

        @pl.when(jnp.logical_not(even))
        def _():
            pltpu.make_async_copy(gtab_hbm.at[srcf.at[pl.ds(boff, BB)]],
                                  buf1, g1).wait()
            scale_rows(buf1, b)
            pltpu.async_copy(buf1, acc.at[dstv.at[b]], s1, add=True)

        return carry

    lax.fori_loop(0, NB, body, 0)
    pltpu.make_async_copy(buf0, acc.at[dstv.at[NB - 1]], s0).wait()
    plsc.subcore_barrier()
    pltpu.sync_copy(esv, esum_hbm.at[w])
    pltpu.sync_copy(exv, ex_hbm.at[w])
    for t in range(5):
        pltpu.sync_copy(acc.at[pl.ds(base + t * BB, BB)], buf0)
        pltpu.sync_copy(buf0, out_hbm.at[c].at[pl.ds(base + t * BB, BB)])


_gat_call = functools.partial(
    pl.kernel,
    out_type=(
        jax.ShapeDtypeStruct((NC, N_TAB, O), jnp.float32),
        jax.ShapeDtypeStruct((NW, N_TAB), jnp.float32),
        jax.ShapeDtypeStruct((NW, EPT), jnp.float32),
    ),
    mesh=_mesh,
    compiler_params=pltpu.CompilerParams(
        needs_layout_passes=False, use_tc_tiling_on_sc=False),
    scratch_types=[
        pltpu.VMEM((EPT,), jnp.int32),
        pltpu.VMEM((EPT,), jnp.int32),
        pltpu.VMEM((NB, BB), jnp.int32),
        pltpu.VMEM((N_TAB,), jnp.float32),
        pltpu.VMEM((N_TAB,), jnp.float32),
        pltpu.VMEM((N_TAB,), jnp.float32),
        pltpu.VMEM((EPT,), jnp.float32),
        pltpu.VMEM((BB, O), jnp.float32),
        pltpu.VMEM((BB, O), jnp.float32),
        pltpu.SemaphoreType.DMA,
        pltpu.SemaphoreType.DMA,
        pltpu.SemaphoreType.DMA,
        pltpu.SemaphoreType.DMA,
        pltpu.VMEM((L,), jnp.float32),
        pltpu.VMEM_SHARED((N_TAB, O), jnp.float32),
    ],
)(_gat_body)


def _alpha_body(ex_hbm, dstf_hbm, rec_hbm, out_hbm, exv, dstf, recv, av):
    w = lax.axis_index("s") * NC + lax.axis_index("c")
    pltpu.sync_copy(ex_hbm.at[w], exv)
    pltpu.sync_copy(dstf_hbm.at[w], dstf)
    pltpu.sync_copy(rec_hbm, recv)

    def body(i, carry):
        off = i * L
        didx = dstf[pl.ds(off, L)]
        r = plsc.load_gather(recv, [didx])
        av[pl.ds(off, L)] = exv[pl.ds(off, L)] * r
        return carry

    lax.fori_loop(0, EPT // L, body, 0)
    pltpu.sync_copy(av, out_hbm.at[w])


_alpha_call = functools.partial(
    pl.kernel,
    out_type=jax.ShapeDtypeStruct((NW, EPT), jnp.float32),
    mesh=_mesh,
    compiler_params=pltpu.CompilerParams(needs_layout_passes=False),
    scratch_types=[
        pltpu.VMEM((EPT,), jnp.float32),
        pltpu.VMEM((EPT,), jnp.int32),
        pltpu.VMEM((N_TAB,), jnp.float32),
        pltpu.VMEM((EPT,), jnp.float32),
    ],
)(_alpha_body)


def _t1_body(x_ref, w1_ref, degp_ref, h0p_ref, dinv_ref):
    deg = degp_ref[0, :] + degp_ref[1, :] + 1.0
    dinv = lax.rsqrt(deg)[:, None]
    h0 = jnp.dot(x_ref[...], w1_ref[...], preferred_element_type=jnp.float32)
    h0p_ref[pl.ds(0, N), :] = h0 * dinv[:N]
    h0p_ref[pl.ds(N, N_TAB - N), :] = jnp.zeros((N_TAB - N, F), jnp.float32)
    dinv_ref[...] = dinv


def _t1(x, W1, degp):
    return pl.pallas_call(
        _t1_body,
        out_shape=[
            jax.ShapeDtypeStruct((N_TAB, F), jnp.float32),
            jax.ShapeDtypeStruct((N_TAB, 1), jnp.float32),
        ],
    )(x, W1, degp)


def _t2_body(p_ref, h0p_ref, dinv_ref, b1_ref, w2_ref, as_ref,
             ad_ref, g_ref, asrc_ref, adst_ref, exs_ref, maxs_ref):
    S = p_ref[0] + p_ref[1] + h0p_ref[...]
    h = jnp.maximum(dinv_ref[...] * S + b1_ref[...], 0.0)
    g = jnp.dot(h, w2_ref[...], preferred_element_type=jnp.float32)
    asrc = jnp.dot(g, as_ref[...], preferred_element_type=jnp.float32)
    adst = jnp.dot(g, ad_ref[...], preferred_element_type=jnp.float32)
    maxs = jnp.max(asrc)
    m = maxs + adst
    m = jnp.where(m >= 0.0, m, m * 0.2)
    e = asrc + adst
    e = jnp.where(e >= 0.0, e, e * 0.2)
    g_ref[...] = g
    asrc_ref[...] = asrc[:, 0]
    adst_ref[...] = adst[:, 0]
    exs_ref[...] = jnp.exp(e - m)
    maxs_ref[...] = jnp.full((1, L), maxs, jnp.float32)


def _t2(p, h0p, dinv, b1, W2, att_src, att_dst):
    return pl.pallas_call(
        _t2_body,
        out_shape=[
            jax.ShapeDtypeStruct((N_TAB, O), jnp.float32),
            jax.ShapeDtypeStruct((N_TAB,), jnp.float32),
            jax.ShapeDtypeStruct((N_TAB,), jnp.float32),
            jax.ShapeDtypeStruct((N_TAB, 1), jnp.float32),
            jax.ShapeDtypeStruct((1, L), jnp.float32),
        ],
    )(p, h0p, dinv, b1, W2, att_src, att_dst)


def _t3_body(pg_ref, esump_ref, exs_ref, g_ref, b2_ref,
             out_ref, aself_ref, rec_ref):
    exs = exs_ref[...]
    esum = jnp.sum(esump_ref[...], axis=0)[:, None] + exs
    rec = 1.0 / (esum + 1e-16)
    g = g_ref[...]
    full = (pg_ref[0] + pg_ref[1] + exs * g) * rec + b2_ref[...]
    out_ref[...] = full[:N]
    aself_ref[...] = (exs * rec)[:N, 0]
    rec_ref[...] = rec[:, 0]


def _t3(pg, esump, exs, g, b2):
    return pl.pallas_call(
        _t3_body,
        out_shape=[
            jax.ShapeDtypeStruct((N, O), jnp.float32),
            jax.ShapeDtypeStruct((N,), jnp.float32),
            jax.ShapeDtypeStruct((N_TAB,), jnp.float32),
        ],
    )(pg, esump, exs, g, b2)


def kernel(x, edge_index, W1, b1, W2, att_src, att_dst, b2):
    src = edge_index[0]
    dst = edge_index[1]
    pad = jnp.full((E_PAD - E,), N, jnp.int32)
    pad_f = jnp.full((E_PAD - E,), N + 1, jnp.int32)
    srcp = jnp.concatenate([src, pad])
    dstp = jnp.concatenate([dst, pad])
    srcf = jnp.concatenate([src, pad_f]).reshape(NW, EPT)
    dstf = jnp.concatenate([dst, pad_f]).reshape(NW, EPT)
    src2 = srcp.reshape(NW, NB, BB)
    dst2 = dstp.reshape(NW, NB, BB)
    zeros_nt = jnp.zeros((N_TAB,), jnp.float32)
    zrow_f = jnp.zeros((BB, FH), jnp.float32)
    zrow_o = jnp.zeros((BB, O), jnp.float32)

    degp = _deg_call(dstf, zeros_nt)
    h0p, dinv = _t1(x, W1, degp)
    P = _gcn_call(h0p, src2, dst2, zrow_f)
    g, asrc, adst, exs, maxs = _t2(P, h0p, dinv, b1.reshape(1, F), W2,
                                   att_src.reshape(O, 1), att_dst.reshape(O, 1))
    Pg, esump, exe = _gat_call(
        g, asrc, adst, maxs.reshape(L),
        srcf, dstf, dst2, zeros_nt, zrow_o)
    out, aself, rec = _t3(Pg, esump, exs, g, b2.reshape(1, O))
    alpha_e = _alpha_call(exe, dstf, rec).reshape(E_PAD)[:E]
    alpha = jnp.concatenate([alpha_e, aself])
    ar = jnp.arange(N, dtype=edge_index.dtype)
    ei_full = jnp.stack([jnp.concatenate([src, ar]), jnp.concatenate([dst, ar])])
    return (out, (ei_full, alpha))
